# baseline (speedup 1.0000x reference)
import functools

import jax
import jax.numpy as jnp
from jax import lax
from jax.experimental import pallas as pl
from jax.experimental.pallas import tpu as pltpu
from jax.experimental.pallas import tpu_sc as plsc

B, S, D = 4, 8192, 4096
NUM_CORES = 2
NUM_SUBCORES = 16
NUM_WORKERS = NUM_CORES * NUM_SUBCORES
CHUNK = (B * D) // NUM_WORKERS
CHUNKS_PER_BATCH = D // CHUNK


@functools.partial(
    pl.kernel,
    out_type=jax.ShapeDtypeStruct((B, 1, D), jnp.float32),
    mesh=plsc.VectorSubcoreMesh(core_axis_name="c", subcore_axis_name="s"),
)
def _last_pool_sc(hid_hbm, out_hbm):
    wid = lax.axis_index("s") * NUM_CORES + lax.axis_index("c")
    b = wid // CHUNKS_PER_BATCH
    off = (wid % CHUNKS_PER_BATCH) * CHUNK
    pltpu.sync_copy(hid_hbm.at[b, 0, pl.ds(off, CHUNK)],
                    out_hbm.at[b, 0, pl.ds(off, CHUNK)])


def kernel(hidden_state):
    return _last_pool_sc(hidden_state)


# speedup vs baseline: 1.0542x; 1.0542x over previous
import functools

import jax
import jax.numpy as jnp
from jax import lax
from jax.experimental import pallas as pl
from jax.experimental.pallas import tpu as pltpu
from jax.experimental.pallas import tpu_sc as plsc

B, S, D = 4, 8192, 4096
NUM_CORES = 2
NUM_SUBCORES = 16
NUM_WORKERS = NUM_CORES * NUM_SUBCORES
CHUNK = (B * D) // NUM_WORKERS
CHUNKS_PER_BATCH = D // CHUNK


@functools.partial(
    pl.kernel,
    out_type=jax.ShapeDtypeStruct((B, 1, D), jnp.float32),
    mesh=plsc.ScalarSubcoreMesh(axis_name="c", num_cores=NUM_CORES),
)
def _last_pool_sc(hid_hbm, out_hbm):
    cid = lax.axis_index("c")
    for i in range(B // NUM_CORES):
        b = cid * (B // NUM_CORES) + i
        pltpu.sync_copy(hid_hbm.at[b, 0, :], out_hbm.at[b, 0, :])


def kernel(hidden_state):
    return _last_pool_sc(hidden_state)


# speedup vs baseline: 6.9039x; 6.5487x over previous
import jax
import jax.numpy as jnp
from jax.experimental import pallas as pl

B, S, D = 4, 8192, 4096


def _body(x_ref, o_ref):
    o_ref[...] = x_ref[:, 0:1, :]


def kernel(hidden_state):
    return pl.pallas_call(
        _body,
        grid=(B,),
        in_specs=[pl.BlockSpec((1, 8, D), lambda b: (b, 0, 0))],
        out_specs=pl.BlockSpec((1, 1, D), lambda b: (b, 0, 0)),
        out_shape=jax.ShapeDtypeStruct((B, 1, D), jnp.float32),
    )(hidden_state)
